# Initial kernel scaffold; baseline (speedup 1.0000x reference)
import jax
import jax.numpy as jnp
from jax.experimental import pallas as pl


def kernel(inputs, table, W, b):
    raise NotImplementedError("write your pallas kernel here")


# speedup vs baseline: 9.0242x; 9.0242x over previous
import functools

import jax
import jax.numpy as jnp
from jax import lax
from jax.experimental import pallas as pl
from jax.experimental.pallas import tpu as pltpu
from jax.experimental.pallas import tpu_sc as plsc


def _pool_kernel_body(CPW, CHUNK_B, S, E, table_hbm, idx_hbm, out_hbm,
                      idx_v, rows_v, pool_v, sem):
    NC = 2
    CHUNK_IDX = CHUNK_B * S
    BPW = CPW * CHUNK_B
    wid = lax.axis_index("s") * NC + lax.axis_index("c")

    pltpu.sync_copy(idx_hbm.at[wid], idx_v)

    pltpu.async_copy(table_hbm.at[idx_v.at[0]], rows_v.at[pl.ds(0, CHUNK_IDX)], sem)
    pltpu.async_copy(table_hbm.at[idx_v.at[1]], rows_v.at[pl.ds(CHUNK_IDX, CHUNK_IDX)], sem)

    nvec = E // 16

    def chunk_body(c, carry):
        par = lax.rem(c, 2)
        base = par * CHUNK_IDX
        pltpu.make_async_copy(
            table_hbm.at[idx_v.at[c]], rows_v.at[pl.ds(0, CHUNK_IDX)], sem
        ).wait()

        @pl.when(c + 2 < CPW)
        def _():
            pltpu.async_copy(
                table_hbm.at[idx_v.at[c + 2]],
                rows_v.at[pl.ds(base, CHUNK_IDX)],
                sem,
            )

        for bag in range(CHUNK_B):
            accs = [jnp.zeros((16,), jnp.float32) for _ in range(nvec)]
            for s in range(S):
                r = base + bag * S + s
                for j in range(nvec):
                    accs[j] = accs[j] + rows_v[r, pl.ds(j * 16, 16)]
            row_out = c * CHUNK_B + bag
            for j in range(nvec):
                pool_v[row_out, pl.ds(j * 16, 16)] = accs[j] * (1.0 / S)
        return carry

    lax.fori_loop(0, CPW, chunk_body, 0, unroll=1)

    pltpu.sync_copy(pool_v, out_hbm.at[pl.ds(wid * BPW, BPW)])


def _make_pool_kernel(B, S, E, NW, CHUNK_B):
    BPW = B // NW
    CPW = BPW // CHUNK_B
    CHUNK_IDX = CHUNK_B * S
    mesh = plsc.VectorSubcoreMesh(core_axis_name="c", subcore_axis_name="s")
    return pl.kernel(
        functools.partial(_pool_kernel_body, CPW, CHUNK_B, S, E),
        out_type=jax.ShapeDtypeStruct((B, E), jnp.float32),
        mesh=mesh,
        scratch_types=[
            pltpu.VMEM((CPW, CHUNK_IDX), jnp.int32),
            pltpu.VMEM((2 * CHUNK_IDX, E), jnp.float32),
            pltpu.VMEM((BPW, E), jnp.float32),
            pltpu.SemaphoreType.DMA,
        ],
        compiler_params=pltpu.CompilerParams(use_tc_tiling_on_sc=False),
    )


def _mm_body(x_ref, w_ref, b_ref, o_ref):
    o_ref[...] = (
        jnp.dot(x_ref[...], w_ref[...], preferred_element_type=jnp.float32)
        + b_ref[...]
    )


def kernel(inputs, table, W, b):
    B, S = inputs.shape
    V, E = table.shape
    O = W.shape[0]
    NW = 32
    CHUNK_B = 2
    BPW = B // NW
    CPW = BPW // CHUNK_B

    idx3 = inputs.astype(jnp.int32).reshape(NW, CPW, CHUNK_B * S)
    pooled = _make_pool_kernel(B, S, E, NW, CHUNK_B)(table, idx3)

    out = pl.pallas_call(
        _mm_body,
        out_shape=jax.ShapeDtypeStruct((B, O), jnp.float32),
    )(pooled, W.T, b.reshape(1, O))
    return out
